# baseline (speedup 1.0000x reference)
import functools

import jax
import jax.numpy as jnp
from jax import lax
from jax.experimental import pallas as pl
from jax.experimental.pallas import tpu as pltpu
from jax.experimental.pallas import tpu_sc as plsc

NC = 2
NS = 16
L = 16
NW = NC * NS

_GDN = lax.GatherDimensionNumbers(offset_dims=(), collapsed_slice_dims=(0,),
                                  start_index_map=(0,))


def _perm(v, idx):
    return lax.gather(v, idx[:, None], _GDN, (1,),
                      mode=lax.GatherScatterMode.PROMISE_IN_BOUNDS)


def _pre_body(x_ref, w1_ref, a1_ref, p_ref, g_ref):
    xv = x_ref[...]
    p_ref[...] = jnp.dot(xv, w1_ref[...], preferred_element_type=jnp.float32)
    g_ref[...] = jnp.dot(xv, a1_ref[...], preferred_element_type=jnp.float32)


def _q_body(ea_ref, w2_ref, q_ref):
    q_ref[...] = jnp.dot(ea_ref[...], w2_ref[...],
                         preferred_element_type=jnp.float32)


def _dsum_body(d_ref, o_ref):
    o_ref[...] = jnp.sum(d_ref[...], axis=0, keepdims=True)


def _post_body(aggp_ref, dcol_ref, x_ref, wza, wzx, wra, wrx, wha, whx,
               bz, br, bh, o_ref):
    agg = (aggp_ref[0] + aggp_ref[1]) / (dcol_ref[...] + 1e-9)
    xv = x_ref[...]
    z = jax.nn.sigmoid(
        jnp.dot(agg, wza[...], preferred_element_type=jnp.float32)
        + jnp.dot(xv, wzx[...], preferred_element_type=jnp.float32)
        + bz[...])
    r = jax.nn.sigmoid(
        jnp.dot(agg, wra[...], preferred_element_type=jnp.float32)
        + jnp.dot(xv, wrx[...], preferred_element_type=jnp.float32)
        + br[...])
    h = jnp.tanh(
        jnp.dot(agg, wha[...], preferred_element_type=jnp.float32)
        + jnp.dot(r * xv, whx[...], preferred_element_type=jnp.float32)
        + bh[...])
    o_ref[...] = (1.0 - z) * xv + z * h


def _make_edge_kernel(n, e, d, npad, blk):
    epw = e // NW
    nblk = epw // blk
    bpad = ((blk + L - 1) // L) * L
    mesh = plsc.VectorSubcoreMesh(core_axis_name="c", subcore_axis_name="s",
                                  num_cores=NC, num_subcores=NS)

    @functools.partial(
        pl.kernel,
        out_type=[
            jax.ShapeDtypeStruct((NC, npad, d), jnp.float32),
            jax.ShapeDtypeStruct((NC, NS, npad), jnp.float32),
        ],
        mesh=mesh,
        compiler_params=pltpu.CompilerParams(needs_layout_passes=False,
                                             use_tc_tiling_on_sc=False),
        scratch_types=[
            pltpu.VMEM((blk,), jnp.int32),
            pltpu.VMEM((1, bpad), jnp.int32),
            pltpu.VMEM((bpad, d), jnp.float32),
            pltpu.VMEM((bpad, d), jnp.float32),
            pltpu.VMEM((bpad,), jnp.float32),
            pltpu.VMEM((npad,), jnp.float32),
            pltpu.VMEM((d,), jnp.float32),
            pltpu.VMEM_SHARED((npad, d), jnp.float32),
            pltpu.SemaphoreType.DMA,
            pltpu.SemaphoreType.DMA,
            pltpu.SemaphoreType.DMA,
        ],
    )
    def edge_kernel(p_hbm, q_hbm, g_hbm, src_hbm, dst_hbm, a2_hbm,
                    agg_hbm, dn_hbm,
                    idx_s, idx_d, q_v, rows_v, gd_v, dn_v, a2_v,
                    agg_sh, sem1, sem2, sem3):
        cid = lax.axis_index("c")
        sid = lax.axis_index("s")
        wid = sid * NC + cid

        pltpu.sync_copy(a2_hbm, a2_v)

        zvec = jnp.zeros((L,), jnp.float32)
        lane = lax.iota(jnp.int32, L)

        def zdn(i, c):
            dn_v[pl.ds(i * L, L)] = zvec
            return c
        lax.fori_loop(0, npad // L, zdn, 0)

        def zrow(rr, c):
            for k in range(d // L):
                rows_v[rr, pl.ds(k * L, L)] = zvec
            return c
        lax.fori_loop(0, bpad, zrow, 0)

        rows_per_tile = npad // NS
        for c in range(rows_per_tile // blk):
            pltpu.sync_copy(rows_v.at[pl.ds(0, blk)],
                            agg_sh.at[pl.ds(sid * rows_per_tile
                                            + c * blk, blk)])
        plsc.subcore_barrier()

        a2v = [a2_v[pl.ds(k * L, L)] for k in range(d // L)]
        bfly = [lane ^ sh for sh in (8, 4, 2, 1)]
        zero_i = lane * 0
        shdn = [jnp.maximum(lane - sh, 0) for sh in (1, 2, 4, 8)]
        nxt = jnp.minimum(lane + 1, L - 1)

        def blk_body(b, carry):
            base = wid * epw + b * blk
            pltpu.sync_copy(src_hbm.at[pl.ds(base, blk)], idx_s)
            pltpu.sync_copy(dst_hbm.at[pl.ds(base, blk)],
                            idx_d.at[0, pl.ds(0, blk)])
            cp1 = pltpu.async_copy(p_hbm.at[idx_s], rows_v.at[pl.ds(0, blk)],
                                   sem1)
            cp2 = pltpu.async_copy(q_hbm.at[pl.ds(base, blk)],
                                   q_v.at[pl.ds(0, blk)], sem2)
            cp3 = pltpu.async_copy(g_hbm.at[idx_d.at[0, pl.ds(0, blk)]],
                                   gd_v.at[pl.ds(0, blk)], sem3)
            cp1.wait()
            cp2.wait()
            cp3.wait()

            def grp_body(gi, c2):
                row0 = gi * L
                gdv = gd_v[pl.ds(row0, L)]
                wgrp = zvec
                for j in range(L):
                    row = row0 + j
                    ek = []
                    acc = None
                    for k in range(d // L):
                        v = rows_v[row, pl.ds(k * L, L)] \
                            + q_v[row, pl.ds(k * L, L)]
                        v = jnp.maximum(v, 0.0)
                        ek.append(v)
                        acc = v * a2v[k] if acc is None else acc + v * a2v[k]
                    for bf in bfly:
                        acc = acc + _perm(acc, bf)
                    gj = _perm(gdv, zero_i + j)
                    lg = gj + acc
                    lg = jnp.where(lg >= 0.0, lg, 0.2 * lg)
                    wv = jnp.exp(lg)
                    for k in range(d // L):
                        rows_v[row, pl.ds(k * L, L)] = ek[k] * wv
                    wgrp = jnp.where(lane == j, wv, wgrp)
                dvec = idx_d[0, pl.ds(row0, L)]
                w = wgrp
                ds_s, ws_s = plsc.sort_key_val(dvec, w)
                for i, sh in enumerate((1, 2, 4, 8)):
                    sd = _perm(ds_s, shdn[i])
                    sw = _perm(ws_s, shdn[i])
                    take = (lane >= sh) & (sd == ds_s)
                    ws_s = ws_s + jnp.where(take, sw, 0.0)
                nd = _perm(ds_s, nxt)
                last = (ds_s != nd) | (lane == L - 1)
                plsc.addupdate_scatter(dn_v, [ds_s], ws_s, mask=last)
                return c2
            lax.fori_loop(0, bpad // L, grp_body, 0)
            pltpu.sync_copy(rows_v, agg_sh.at[idx_d.at[0]], add=True)
            return carry
        lax.fori_loop(0, nblk, blk_body, 0)

        plsc.subcore_barrier()
        for c in range(rows_per_tile // blk):
            r0 = sid * rows_per_tile + c * blk
            pltpu.sync_copy(agg_sh.at[pl.ds(r0, blk)],
                            rows_v.at[pl.ds(0, blk)])
            pltpu.sync_copy(rows_v.at[pl.ds(0, blk)],
                            agg_hbm.at[cid, pl.ds(r0, blk)])
        pltpu.sync_copy(dn_v, dn_hbm.at[cid, sid])

    return edge_kernel


def kernel(x, edge_attr, W_edge, a_att, W_z, b_z, W_r, b_r, W_h, b_h,
           edge_index):
    n, d = x.shape
    e, de = edge_attr.shape
    blk = 80
    align = NS * blk
    npad = ((n + align - 1) // align) * align
    rblk = 400
    qblk = 4000

    src = edge_index[0].astype(jnp.int32)
    dst = edge_index[1].astype(jnp.int32)
    w1 = W_edge[:d]
    w2 = W_edge[d:]
    a1 = a_att[:d]
    a2 = a_att[d:, 0]

    p_mat, g_mat = pl.pallas_call(
        _pre_body,
        grid=(n // rblk,),
        in_specs=[
            pl.BlockSpec((rblk, d), lambda i: (i, 0)),
            pl.BlockSpec((d, d), lambda i: (0, 0)),
            pl.BlockSpec((d, 1), lambda i: (0, 0)),
        ],
        out_specs=[
            pl.BlockSpec((rblk, d), lambda i: (i, 0)),
            pl.BlockSpec((rblk, 1), lambda i: (i, 0)),
        ],
        out_shape=[
            jax.ShapeDtypeStruct((n, d), jnp.float32),
            jax.ShapeDtypeStruct((n, 1), jnp.float32),
        ],
    )(x, w1, a1)
    g_vec = g_mat.reshape(n)

    q_mat = pl.pallas_call(
        _q_body,
        grid=(e // qblk,),
        in_specs=[
            pl.BlockSpec((qblk, de), lambda i: (i, 0)),
            pl.BlockSpec((de, d), lambda i: (0, 0)),
        ],
        out_specs=pl.BlockSpec((qblk, d), lambda i: (i, 0)),
        out_shape=jax.ShapeDtypeStruct((e, d), jnp.float32),
    )(edge_attr, w2)

    edge_kernel = _make_edge_kernel(n, e, d, npad, blk)
    agg_parts, dn_parts = edge_kernel(p_mat, q_mat, g_vec, src, dst, a2)

    dsum = pl.pallas_call(
        _dsum_body,
        in_specs=[pl.BlockSpec((NW, npad), lambda: (0, 0))],
        out_specs=pl.BlockSpec((1, npad), lambda: (0, 0)),
        out_shape=jax.ShapeDtypeStruct((1, npad), jnp.float32),
    )(dn_parts.reshape(NW, npad))
    dcol = dsum.reshape(npad)[:n].reshape(n, 1)

    out = pl.pallas_call(
        _post_body,
        grid=(n // rblk,),
        in_specs=[
            pl.BlockSpec((NC, rblk, d), lambda i: (0, i, 0)),
            pl.BlockSpec((rblk, 1), lambda i: (i, 0)),
            pl.BlockSpec((rblk, d), lambda i: (i, 0)),
            pl.BlockSpec((d, d), lambda i: (0, 0)),
            pl.BlockSpec((d, d), lambda i: (0, 0)),
            pl.BlockSpec((d, d), lambda i: (0, 0)),
            pl.BlockSpec((d, d), lambda i: (0, 0)),
            pl.BlockSpec((d, d), lambda i: (0, 0)),
            pl.BlockSpec((d, d), lambda i: (0, 0)),
            pl.BlockSpec((1, d), lambda i: (0, 0)),
            pl.BlockSpec((1, d), lambda i: (0, 0)),
            pl.BlockSpec((1, d), lambda i: (0, 0)),
        ],
        out_specs=pl.BlockSpec((rblk, d), lambda i: (i, 0)),
        out_shape=jax.ShapeDtypeStruct((n, d), jnp.float32),
    )(agg_parts, dcol, x, W_z[:d], W_z[d:], W_r[:d], W_r[d:], W_h[:d],
      W_h[d:], b_z.reshape(1, d), b_r.reshape(1, d), b_h.reshape(1, d))
    return out


# speedup vs baseline: 1.0295x; 1.0295x over previous
import functools

import jax
import jax.numpy as jnp
from jax import lax
from jax.experimental import pallas as pl
from jax.experimental.pallas import tpu as pltpu
from jax.experimental.pallas import tpu_sc as plsc

NC = 2
NS = 16
L = 16
NW = NC * NS

_GDN = lax.GatherDimensionNumbers(offset_dims=(), collapsed_slice_dims=(0,),
                                  start_index_map=(0,))


def _perm(v, idx):
    return lax.gather(v, idx[:, None], _GDN, (1,),
                      mode=lax.GatherScatterMode.PROMISE_IN_BOUNDS)


def _pre_body(x_ref, w1_ref, a1_ref, p_ref, g_ref):
    xv = x_ref[...]
    p_ref[...] = jnp.dot(xv, w1_ref[...], preferred_element_type=jnp.float32)
    g_ref[...] = jnp.dot(xv, a1_ref[...], preferred_element_type=jnp.float32)


def _q_body(ea_ref, w2_ref, q_ref):
    q_ref[...] = jnp.dot(ea_ref[...], w2_ref[...],
                         preferred_element_type=jnp.float32)


def _dsum_body(d_ref, o_ref):
    o_ref[...] = jnp.sum(d_ref[...], axis=0, keepdims=True)


def _post_body(aggp_ref, dcol_ref, x_ref, wza, wzx, wra, wrx, wha, whx,
               bz, br, bh, o_ref):
    agg = (aggp_ref[0] + aggp_ref[1]) / (dcol_ref[...] + 1e-9)
    xv = x_ref[...]
    z = jax.nn.sigmoid(
        jnp.dot(agg, wza[...], preferred_element_type=jnp.float32)
        + jnp.dot(xv, wzx[...], preferred_element_type=jnp.float32)
        + bz[...])
    r = jax.nn.sigmoid(
        jnp.dot(agg, wra[...], preferred_element_type=jnp.float32)
        + jnp.dot(xv, wrx[...], preferred_element_type=jnp.float32)
        + br[...])
    h = jnp.tanh(
        jnp.dot(agg, wha[...], preferred_element_type=jnp.float32)
        + jnp.dot(r * xv, whx[...], preferred_element_type=jnp.float32)
        + bh[...])
    o_ref[...] = (1.0 - z) * xv + z * h


def _make_edge_kernel(n, e, d, npad, blk):
    epw = e // NW
    nblk = epw // blk
    bpad = ((blk + L - 1) // L) * L
    mesh = plsc.VectorSubcoreMesh(core_axis_name="c", subcore_axis_name="s",
                                  num_cores=NC, num_subcores=NS)

    nhalf = nblk // 2

    @functools.partial(
        pl.kernel,
        out_type=[
            jax.ShapeDtypeStruct((NC, npad, d), jnp.float32),
            jax.ShapeDtypeStruct((NC, NS, npad), jnp.float32),
        ],
        mesh=mesh,
        compiler_params=pltpu.CompilerParams(needs_layout_passes=False,
                                             use_tc_tiling_on_sc=False),
        scratch_types=[
            pltpu.VMEM((2, blk), jnp.int32),
            pltpu.VMEM((2, bpad), jnp.int32),
            pltpu.VMEM((2, bpad), jnp.int32),
            pltpu.VMEM((2, bpad, d), jnp.float32),
            pltpu.VMEM((2, bpad, d), jnp.float32),
            pltpu.VMEM((2, bpad), jnp.float32),
            pltpu.VMEM((npad,), jnp.float32),
            pltpu.VMEM((d,), jnp.float32),
            pltpu.VMEM_SHARED((npad, d), jnp.float32),
            pltpu.SemaphoreType.DMA,
            pltpu.SemaphoreType.DMA,
            pltpu.SemaphoreType.DMA,
            pltpu.SemaphoreType.DMA,
            pltpu.SemaphoreType.DMA,
            pltpu.SemaphoreType.DMA,
        ],
    )
    def edge_kernel(p_hbm, q_hbm, g_hbm, src_hbm, dst_hbm, a2_hbm,
                    agg_hbm, dn_hbm,
                    idx_s, idx_d, idxsc, q_v, rows_v, gd_v, dn_v, a2_v,
                    agg_sh, sg0, sg1, si0, si1, ss0, ss1):
        cid = lax.axis_index("c")
        sid = lax.axis_index("s")
        wid = sid * NC + cid
        sg = (sg0, sg1)
        si = (si0, si1)
        ss = (ss0, ss1)

        pltpu.sync_copy(a2_hbm, a2_v)

        zvec = jnp.zeros((L,), jnp.float32)
        lane = lax.iota(jnp.int32, L)

        def zdn(i, c):
            dn_v[pl.ds(i * L, L)] = zvec
            return c
        lax.fori_loop(0, npad // L, zdn, 0)

        def zrow(rr, c):
            for o in (0, 1):
                for k in range(d // L):
                    rows_v[o, rr, pl.ds(k * L, L)] = zvec
                    q_v[o, rr, pl.ds(k * L, L)] = zvec
            return c
        lax.fori_loop(0, bpad, zrow, 0)
        for o in (0, 1):
            for kk in range(bpad // L):
                gd_v[o, pl.ds(kk * L, L)] = zvec
            if bpad > blk:
                tail0 = (bpad // L - 1) * L
                idx_d[o, pl.ds(tail0, L)] = jnp.where(lane >= blk - tail0,
                                                      npad - 1, 0)

        rows_per_tile = npad // NS
        for c in range(rows_per_tile // blk):
            pltpu.sync_copy(rows_v.at[0, pl.ds(0, blk)],
                            agg_sh.at[pl.ds(sid * rows_per_tile
                                            + c * blk, blk)])
        plsc.subcore_barrier()

        a2v = [a2_v[pl.ds(k * L, L)] for k in range(d // L)]
        bfly = [lane ^ sh for sh in (8, 4, 2, 1)]
        zero_i = lane * 0
        shdn = [jnp.maximum(lane - sh, 0) for sh in (1, 2, 4, 8)]
        nxt = jnp.minimum(lane + 1, L - 1)

        def base_of(t):
            return wid * epw + t * blk

        def idx_descs(t, o, sem):
            b = base_of(t)
            return (
                pltpu.make_async_copy(src_hbm.at[pl.ds(b, blk)],
                                      idx_s.at[o], sem),
                pltpu.make_async_copy(dst_hbm.at[pl.ds(b, blk)],
                                      idx_d.at[o, pl.ds(0, blk)], sem),
            )

        def gather_descs(t, o, sem):
            b = base_of(t)
            return (
                pltpu.make_async_copy(p_hbm.at[idx_s.at[o]],
                                      rows_v.at[o, pl.ds(0, blk)], sem),
                pltpu.make_async_copy(q_hbm.at[pl.ds(b, blk)],
                                      q_v.at[o, pl.ds(0, blk)], sem),
                pltpu.make_async_copy(g_hbm.at[idx_d.at[o, pl.ds(0, blk)]],
                                      gd_v.at[o, pl.ds(0, blk)], sem),
            )

        def issue_idx(t, o):
            for cp in idx_descs(t, o, si[o]):
                cp.start()

        def drain_idx(t, o):
            for cp in idx_descs(t, o, si[o]):
                cp.wait()

        def issue_gathers(t, o):
            for cp in gather_descs(t, o, sg[o]):
                cp.start()

        def drain_gathers(t, o):
            for cp in gather_descs(t, o, sg[o]):
                cp.wait()

        def issue_scatter(s):
            pltpu.async_copy(rows_v.at[s], agg_sh.at[idxsc.at[s]], ss[s],
                             add=True)

        def drain_scatter(s):
            pltpu.make_async_copy(rows_v.at[s], agg_sh.at[idxsc.at[s]],
                                  ss[s]).wait()

        def compute_block(s):
            def grp_body(gi, c2):
                row0 = gi * L
                svec = zvec
                for j in range(L):
                    row = row0 + j
                    acc = None
                    for k in range(d // L):
                        v = rows_v[s, row, pl.ds(k * L, L)] \
                            + q_v[s, row, pl.ds(k * L, L)]
                        v = jnp.maximum(v, 0.0)
                        rows_v[s, row, pl.ds(k * L, L)] = v
                        acc = v * a2v[k] if acc is None else acc + v * a2v[k]
                    for bf in bfly:
                        acc = acc + _perm(acc, bf)
                    svec = jnp.where(lane == j, acc, svec)
                dvec = idx_d[s, pl.ds(row0, L)]
                gd = gd_v[s, pl.ds(row0, L)]
                logit = gd + svec
                logit = jnp.where(logit >= 0.0, logit, 0.2 * logit)
                w = jnp.exp(logit)
                for j in range(L):
                    row = row0 + j
                    wj = _perm(w, zero_i + j)
                    for k in range(d // L):
                        rows_v[s, row, pl.ds(k * L, L)] = \
                            rows_v[s, row, pl.ds(k * L, L)] * wj
                ds_s, ws_s = plsc.sort_key_val(dvec, w)
                for i, sh in enumerate((1, 2, 4, 8)):
                    sd = _perm(ds_s, shdn[i])
                    sw = _perm(ws_s, shdn[i])
                    take = (lane >= sh) & (sd == ds_s)
                    ws_s = ws_s + jnp.where(take, sw, 0.0)
                nd = _perm(ds_s, nxt)
                last = (ds_s != nd) | (lane == L - 1)
                plsc.addupdate_scatter(dn_v, [ds_s], ws_s, mask=last)
                return c2
            lax.fori_loop(0, bpad // L, grp_body, 0)
            for kk in range(bpad // L):
                idxsc[s, pl.ds(kk * L, L)] = idx_d[s, pl.ds(kk * L, L)]

        b0 = base_of(0)
        pltpu.sync_copy(src_hbm.at[pl.ds(b0, blk)], idx_s.at[0])
        pltpu.sync_copy(dst_hbm.at[pl.ds(b0, blk)],
                        idx_d.at[0, pl.ds(0, blk)])
        issue_gathers(0, 0)

        def pipe_body(t2, carry):
            t0 = 2 * t2
            drain_gathers(t0, 0)
            issue_idx(t0 + 1, 1)
            compute_block(0)
            issue_scatter(0)
            drain_idx(t0 + 1, 1)

            @pl.when(t2 > 0)
            def _():
                drain_scatter(1)
            issue_gathers(t0 + 1, 1)

            t1 = t0 + 1
            drain_gathers(t1, 1)

            @pl.when(t2 < nhalf - 1)
            def _():
                issue_idx(t1 + 1, 0)
            compute_block(1)
            issue_scatter(1)

            @pl.when(t2 < nhalf - 1)
            def _():
                drain_idx(t1 + 1, 0)
                drain_scatter(0)
                issue_gathers(t1 + 1, 0)
            return carry
        lax.fori_loop(0, nhalf, pipe_body, 0)
        drain_scatter(0)
        drain_scatter(1)

        plsc.subcore_barrier()
        for c in range(rows_per_tile // blk):
            r0 = sid * rows_per_tile + c * blk
            pltpu.sync_copy(agg_sh.at[pl.ds(r0, blk)],
                            rows_v.at[0, pl.ds(0, blk)])
            pltpu.sync_copy(rows_v.at[0, pl.ds(0, blk)],
                            agg_hbm.at[cid, pl.ds(r0, blk)])
        pltpu.sync_copy(dn_v, dn_hbm.at[cid, sid])

    return edge_kernel


def kernel(x, edge_attr, W_edge, a_att, W_z, b_z, W_r, b_r, W_h, b_h,
           edge_index):
    n, d = x.shape
    e, de = edge_attr.shape
    blk = 40
    align = NS * blk
    npad = ((n + align - 1) // align) * align
    rblk = 400
    qblk = 4000

    src = edge_index[0].astype(jnp.int32)
    dst = edge_index[1].astype(jnp.int32)
    w1 = W_edge[:d]
    w2 = W_edge[d:]
    a1 = a_att[:d]
    a2 = a_att[d:, 0]

    p_mat, g_mat = pl.pallas_call(
        _pre_body,
        grid=(n // rblk,),
        in_specs=[
            pl.BlockSpec((rblk, d), lambda i: (i, 0)),
            pl.BlockSpec((d, d), lambda i: (0, 0)),
            pl.BlockSpec((d, 1), lambda i: (0, 0)),
        ],
        out_specs=[
            pl.BlockSpec((rblk, d), lambda i: (i, 0)),
            pl.BlockSpec((rblk, 1), lambda i: (i, 0)),
        ],
        out_shape=[
            jax.ShapeDtypeStruct((n, d), jnp.float32),
            jax.ShapeDtypeStruct((n, 1), jnp.float32),
        ],
    )(x, w1, a1)
    g_vec = g_mat.reshape(n)

    q_mat = pl.pallas_call(
        _q_body,
        grid=(e // qblk,),
        in_specs=[
            pl.BlockSpec((qblk, de), lambda i: (i, 0)),
            pl.BlockSpec((de, d), lambda i: (0, 0)),
        ],
        out_specs=pl.BlockSpec((qblk, d), lambda i: (i, 0)),
        out_shape=jax.ShapeDtypeStruct((e, d), jnp.float32),
    )(edge_attr, w2)

    edge_kernel = _make_edge_kernel(n, e, d, npad, blk)
    agg_parts, dn_parts = edge_kernel(p_mat, q_mat, g_vec, src, dst, a2)

    dsum = pl.pallas_call(
        _dsum_body,
        in_specs=[pl.BlockSpec((NW, npad), lambda: (0, 0))],
        out_specs=pl.BlockSpec((1, npad), lambda: (0, 0)),
        out_shape=jax.ShapeDtypeStruct((1, npad), jnp.float32),
    )(dn_parts.reshape(NW, npad))
    dcol = dsum.reshape(npad)[:n].reshape(n, 1)

    out = pl.pallas_call(
        _post_body,
        grid=(n // rblk,),
        in_specs=[
            pl.BlockSpec((NC, rblk, d), lambda i: (0, i, 0)),
            pl.BlockSpec((rblk, 1), lambda i: (i, 0)),
            pl.BlockSpec((rblk, d), lambda i: (i, 0)),
            pl.BlockSpec((d, d), lambda i: (0, 0)),
            pl.BlockSpec((d, d), lambda i: (0, 0)),
            pl.BlockSpec((d, d), lambda i: (0, 0)),
            pl.BlockSpec((d, d), lambda i: (0, 0)),
            pl.BlockSpec((d, d), lambda i: (0, 0)),
            pl.BlockSpec((d, d), lambda i: (0, 0)),
            pl.BlockSpec((1, d), lambda i: (0, 0)),
            pl.BlockSpec((1, d), lambda i: (0, 0)),
            pl.BlockSpec((1, d), lambda i: (0, 0)),
        ],
        out_specs=pl.BlockSpec((rblk, d), lambda i: (i, 0)),
        out_shape=jax.ShapeDtypeStruct((n, d), jnp.float32),
    )(agg_parts, dcol, x, W_z[:d], W_z[d:], W_r[:d], W_r[d:], W_h[:d],
      W_h[d:], b_z.reshape(1, d), b_r.reshape(1, d), b_h.reshape(1, d))
    return out


# speedup vs baseline: 1.0694x; 1.0387x over previous
import functools

import jax
import jax.numpy as jnp
from jax import lax
from jax.experimental import pallas as pl
from jax.experimental.pallas import tpu as pltpu
from jax.experimental.pallas import tpu_sc as plsc

NC = 2
NS = 16
L = 16
NW = NC * NS

_GDN = lax.GatherDimensionNumbers(offset_dims=(), collapsed_slice_dims=(0,),
                                  start_index_map=(0,))


def _perm(v, idx):
    return lax.gather(v, idx[:, None], _GDN, (1,),
                      mode=lax.GatherScatterMode.PROMISE_IN_BOUNDS)


def _pre_body(x_ref, w1_ref, a1_ref, p_ref, g_ref):
    xv = x_ref[...]
    p_ref[...] = jnp.dot(xv, w1_ref[...], preferred_element_type=jnp.float32)
    g_ref[...] = jnp.dot(xv, a1_ref[...], preferred_element_type=jnp.float32)


def _q_body(ea_ref, w2_ref, q_ref):
    q_ref[...] = jnp.dot(ea_ref[...], w2_ref[...],
                         preferred_element_type=jnp.float32)


def _dsum_body(d_ref, o_ref):
    o_ref[...] = jnp.sum(d_ref[...], axis=0, keepdims=True)


def _post_body(aggp_ref, dcol_ref, x_ref, wza, wzx, wra, wrx, wha, whx,
               bz, br, bh, o_ref):
    agg = (aggp_ref[0] + aggp_ref[1]) / (dcol_ref[...] + 1e-9)
    xv = x_ref[...]
    z = jax.nn.sigmoid(
        jnp.dot(agg, wza[...], preferred_element_type=jnp.float32)
        + jnp.dot(xv, wzx[...], preferred_element_type=jnp.float32)
        + bz[...])
    r = jax.nn.sigmoid(
        jnp.dot(agg, wra[...], preferred_element_type=jnp.float32)
        + jnp.dot(xv, wrx[...], preferred_element_type=jnp.float32)
        + br[...])
    h = jnp.tanh(
        jnp.dot(agg, wha[...], preferred_element_type=jnp.float32)
        + jnp.dot(r * xv, whx[...], preferred_element_type=jnp.float32)
        + bh[...])
    o_ref[...] = (1.0 - z) * xv + z * h


def _make_edge_kernel(n, e, d, npad, blk):
    epw = e // NW
    nblk = epw // blk
    bpad = ((blk + L - 1) // L) * L
    mesh = plsc.VectorSubcoreMesh(core_axis_name="c", subcore_axis_name="s",
                                  num_cores=NC, num_subcores=NS)

    @functools.partial(
        pl.kernel,
        out_type=[
            jax.ShapeDtypeStruct((NC, npad, d), jnp.float32),
            jax.ShapeDtypeStruct((NC, NS, npad), jnp.float32),
        ],
        mesh=mesh,
        compiler_params=pltpu.CompilerParams(needs_layout_passes=False,
                                             use_tc_tiling_on_sc=False),
        scratch_types=[
            pltpu.VMEM((blk,), jnp.int32),
            pltpu.VMEM((1, bpad), jnp.int32),
            pltpu.VMEM((bpad, d), jnp.float32),
            pltpu.VMEM((bpad, d), jnp.float32),
            pltpu.VMEM((bpad,), jnp.float32),
            pltpu.VMEM((npad,), jnp.float32),
            pltpu.VMEM((d,), jnp.float32),
            pltpu.VMEM_SHARED((npad, d), jnp.float32),
            pltpu.SemaphoreType.DMA,
            pltpu.SemaphoreType.DMA,
            pltpu.SemaphoreType.DMA,
        ],
    )
    def edge_kernel(p_hbm, q_hbm, g_hbm, src_hbm, dst_hbm, a2_hbm,
                    agg_hbm, dn_hbm,
                    idx_s, idx_d, q_v, rows_v, gd_v, dn_v, a2_v,
                    agg_sh, sem1, sem2, sem3):
        cid = lax.axis_index("c")
        sid = lax.axis_index("s")
        wid = sid * NC + cid

        pltpu.sync_copy(a2_hbm, a2_v)

        zvec = jnp.zeros((L,), jnp.float32)
        lane = lax.iota(jnp.int32, L)

        def zdn(i, c):
            dn_v[pl.ds(i * L, L)] = zvec
            return c
        lax.fori_loop(0, npad // L, zdn, 0)

        def zrow(rr, c):
            for k in range(d // L):
                rows_v[rr, pl.ds(k * L, L)] = zvec
            return c
        lax.fori_loop(0, bpad, zrow, 0)

        rows_per_tile = npad // NS
        for c in range(rows_per_tile // blk):
            pltpu.sync_copy(rows_v.at[pl.ds(0, blk)],
                            agg_sh.at[pl.ds(sid * rows_per_tile
                                            + c * blk, blk)])
        plsc.subcore_barrier()

        a2v = [a2_v[pl.ds(k * L, L)] for k in range(d // L)]
        bfly = [lane ^ sh for sh in (8, 4, 2, 1)]
        zero_i = lane * 0
        shdn = [jnp.maximum(lane - sh, 0) for sh in (1, 2, 4, 8)]
        nxt = jnp.minimum(lane + 1, L - 1)

        def blk_body(b, carry):
            base = wid * epw + b * blk
            pltpu.sync_copy(src_hbm.at[pl.ds(base, blk)], idx_s)
            pltpu.sync_copy(dst_hbm.at[pl.ds(base, blk)],
                            idx_d.at[0, pl.ds(0, blk)])
            cp1 = pltpu.async_copy(p_hbm.at[idx_s], rows_v.at[pl.ds(0, blk)],
                                   sem1)
            cp2 = pltpu.async_copy(q_hbm.at[pl.ds(base, blk)],
                                   q_v.at[pl.ds(0, blk)], sem2)
            cp3 = pltpu.async_copy(g_hbm.at[idx_d.at[0, pl.ds(0, blk)]],
                                   gd_v.at[pl.ds(0, blk)], sem3)
            cp1.wait()
            cp2.wait()
            cp3.wait()

            def grp_body(gi, c2):
                row0 = gi * L
                svec = zvec
                for j in range(L):
                    row = row0 + j
                    acc = None
                    for k in range(d // L):
                        v = rows_v[row, pl.ds(k * L, L)] \
                            + q_v[row, pl.ds(k * L, L)]
                        v = jnp.maximum(v, 0.0)
                        rows_v[row, pl.ds(k * L, L)] = v
                        acc = v * a2v[k] if acc is None else acc + v * a2v[k]
                    for bf in bfly:
                        acc = acc + _perm(acc, bf)
                    svec = jnp.where(lane == j, acc, svec)
                dvec = idx_d[0, pl.ds(row0, L)]
                gd = gd_v[pl.ds(row0, L)]
                logit = gd + svec
                logit = jnp.where(logit >= 0.0, logit, 0.2 * logit)
                w = jnp.exp(logit)
                for j in range(L):
                    row = row0 + j
                    wj = _perm(w, zero_i + j)
                    for k in range(d // L):
                        rows_v[row, pl.ds(k * L, L)] = \
                            rows_v[row, pl.ds(k * L, L)] * wj
                ds_s, ws_s = plsc.sort_key_val(dvec, w)
                for i, sh in enumerate((1, 2, 4, 8)):
                    sd = _perm(ds_s, shdn[i])
                    sw = _perm(ws_s, shdn[i])
                    take = (lane >= sh) & (sd == ds_s)
                    ws_s = ws_s + jnp.where(take, sw, 0.0)
                nd = _perm(ds_s, nxt)
                last = (ds_s != nd) | (lane == L - 1)
                plsc.addupdate_scatter(dn_v, [ds_s], ws_s, mask=last)
                return c2
            lax.fori_loop(0, bpad // L, grp_body, 0)
            pltpu.sync_copy(rows_v, agg_sh.at[idx_d.at[0]], add=True)
            return carry
        lax.fori_loop(0, nblk, blk_body, 0)

        plsc.subcore_barrier()
        for c in range(rows_per_tile // blk):
            r0 = sid * rows_per_tile + c * blk
            pltpu.sync_copy(agg_sh.at[pl.ds(r0, blk)],
                            rows_v.at[pl.ds(0, blk)])
            pltpu.sync_copy(rows_v.at[pl.ds(0, blk)],
                            agg_hbm.at[cid, pl.ds(r0, blk)])
        pltpu.sync_copy(dn_v, dn_hbm.at[cid, sid])

    return edge_kernel


def kernel(x, edge_attr, W_edge, a_att, W_z, b_z, W_r, b_r, W_h, b_h,
           edge_index):
    n, d = x.shape
    e, de = edge_attr.shape
    blk = 80
    align = NS * blk
    npad = ((n + align - 1) // align) * align
    rblk = 400
    qblk = 4000

    src = edge_index[0].astype(jnp.int32)
    dst = edge_index[1].astype(jnp.int32)
    w1 = W_edge[:d]
    w2 = W_edge[d:]
    a1 = a_att[:d]
    a2 = a_att[d:, 0]

    p_mat, g_mat = pl.pallas_call(
        _pre_body,
        grid=(n // rblk,),
        in_specs=[
            pl.BlockSpec((rblk, d), lambda i: (i, 0)),
            pl.BlockSpec((d, d), lambda i: (0, 0)),
            pl.BlockSpec((d, 1), lambda i: (0, 0)),
        ],
        out_specs=[
            pl.BlockSpec((rblk, d), lambda i: (i, 0)),
            pl.BlockSpec((rblk, 1), lambda i: (i, 0)),
        ],
        out_shape=[
            jax.ShapeDtypeStruct((n, d), jnp.float32),
            jax.ShapeDtypeStruct((n, 1), jnp.float32),
        ],
    )(x, w1, a1)
    g_vec = g_mat.reshape(n)

    q_mat = pl.pallas_call(
        _q_body,
        grid=(e // qblk,),
        in_specs=[
            pl.BlockSpec((qblk, de), lambda i: (i, 0)),
            pl.BlockSpec((de, d), lambda i: (0, 0)),
        ],
        out_specs=pl.BlockSpec((qblk, d), lambda i: (i, 0)),
        out_shape=jax.ShapeDtypeStruct((e, d), jnp.float32),
    )(edge_attr, w2)

    edge_kernel = _make_edge_kernel(n, e, d, npad, blk)
    agg_parts, dn_parts = edge_kernel(p_mat, q_mat, g_vec, src, dst, a2)

    dsum = pl.pallas_call(
        _dsum_body,
        in_specs=[pl.BlockSpec((NW, npad), lambda: (0, 0))],
        out_specs=pl.BlockSpec((1, npad), lambda: (0, 0)),
        out_shape=jax.ShapeDtypeStruct((1, npad), jnp.float32),
    )(dn_parts.reshape(NW, npad))
    dcol = dsum.reshape(npad)[:n].reshape(n, 1)

    out = pl.pallas_call(
        _post_body,
        grid=(n // rblk,),
        in_specs=[
            pl.BlockSpec((NC, rblk, d), lambda i: (0, i, 0)),
            pl.BlockSpec((rblk, 1), lambda i: (i, 0)),
            pl.BlockSpec((rblk, d), lambda i: (i, 0)),
            pl.BlockSpec((d, d), lambda i: (0, 0)),
            pl.BlockSpec((d, d), lambda i: (0, 0)),
            pl.BlockSpec((d, d), lambda i: (0, 0)),
            pl.BlockSpec((d, d), lambda i: (0, 0)),
            pl.BlockSpec((d, d), lambda i: (0, 0)),
            pl.BlockSpec((d, d), lambda i: (0, 0)),
            pl.BlockSpec((1, d), lambda i: (0, 0)),
            pl.BlockSpec((1, d), lambda i: (0, 0)),
            pl.BlockSpec((1, d), lambda i: (0, 0)),
        ],
        out_specs=pl.BlockSpec((rblk, d), lambda i: (i, 0)),
        out_shape=jax.ShapeDtypeStruct((n, d), jnp.float32),
    )(agg_parts, dcol, x, W_z[:d], W_z[d:], W_r[:d], W_r[d:], W_h[:d],
      W_h[d:], b_z.reshape(1, d), b_r.reshape(1, d), b_h.reshape(1, d))
    return out


# speedup vs baseline: 1.1503x; 1.0757x over previous
import functools

import jax
import jax.numpy as jnp
from jax import lax
from jax.experimental import pallas as pl
from jax.experimental.pallas import tpu as pltpu
from jax.experimental.pallas import tpu_sc as plsc

NC = 2
NS = 16
L = 16
NW = NC * NS

_GDN = lax.GatherDimensionNumbers(offset_dims=(), collapsed_slice_dims=(0,),
                                  start_index_map=(0,))


def _perm(v, idx):
    return lax.gather(v, idx[:, None], _GDN, (1,),
                      mode=lax.GatherScatterMode.PROMISE_IN_BOUNDS)


def _pre_body(x_ref, w1_ref, a1_ref, p_ref, g_ref):
    xv = x_ref[...]
    p_ref[...] = jnp.dot(xv, w1_ref[...], preferred_element_type=jnp.float32)
    g_ref[...] = jnp.dot(xv, a1_ref[...], preferred_element_type=jnp.float32)


def _q_body(ea_ref, w2_ref, q_ref):
    q_ref[...] = jnp.dot(ea_ref[...], w2_ref[...],
                         preferred_element_type=jnp.float32)


def _dsum_body(d_ref, o_ref):
    o_ref[...] = jnp.sum(d_ref[...], axis=0, keepdims=True)


def _post_body(aggp_ref, dcol_ref, x_ref, wza, wzx, wra, wrx, wha, whx,
               bz, br, bh, o_ref):
    agg = (aggp_ref[0] + aggp_ref[1]) / (dcol_ref[...] + 1e-9)
    xv = x_ref[...]
    z = jax.nn.sigmoid(
        jnp.dot(agg, wza[...], preferred_element_type=jnp.float32)
        + jnp.dot(xv, wzx[...], preferred_element_type=jnp.float32)
        + bz[...])
    r = jax.nn.sigmoid(
        jnp.dot(agg, wra[...], preferred_element_type=jnp.float32)
        + jnp.dot(xv, wrx[...], preferred_element_type=jnp.float32)
        + br[...])
    h = jnp.tanh(
        jnp.dot(agg, wha[...], preferred_element_type=jnp.float32)
        + jnp.dot(r * xv, whx[...], preferred_element_type=jnp.float32)
        + bh[...])
    o_ref[...] = (1.0 - z) * xv + z * h


def _make_edge_kernel(n, e, d, npad, blk):
    epw = e // NW
    nfull = epw // blk
    tail = epw - nfull * blk
    nhalf = nfull // 2
    assert blk % L == 0 and nfull % 2 == 0 and tail % L == 0
    mesh = plsc.VectorSubcoreMesh(core_axis_name="c", subcore_axis_name="s",
                                  num_cores=NC, num_subcores=NS)

    @functools.partial(
        pl.kernel,
        out_type=[
            jax.ShapeDtypeStruct((NC, npad, d), jnp.float32),
            jax.ShapeDtypeStruct((NC, NS, npad), jnp.float32),
        ],
        mesh=mesh,
        compiler_params=pltpu.CompilerParams(needs_layout_passes=False,
                                             use_tc_tiling_on_sc=False),
        scratch_types=[
            pltpu.VMEM((2, blk), jnp.int32),
            pltpu.VMEM((2, blk), jnp.int32),
            pltpu.VMEM((2, blk), jnp.int32),
            pltpu.VMEM((2, blk, d), jnp.float32),
            pltpu.VMEM((2, blk, d), jnp.float32),
            pltpu.VMEM((2, blk), jnp.float32),
            pltpu.VMEM((npad,), jnp.float32),
            pltpu.VMEM((d,), jnp.float32),
            pltpu.VMEM_SHARED((npad, d), jnp.float32),
            pltpu.SemaphoreType.DMA,
            pltpu.SemaphoreType.DMA,
            pltpu.SemaphoreType.DMA,
            pltpu.SemaphoreType.DMA,
            pltpu.SemaphoreType.DMA,
            pltpu.SemaphoreType.DMA,
        ],
    )
    def edge_kernel(p_hbm, q_hbm, g_hbm, src_hbm, dst_hbm, a2_hbm,
                    agg_hbm, dn_hbm,
                    idx_s, idx_d, idxsc, q_v, rows_v, gd_v, dn_v, a2_v,
                    agg_sh, sg0, sg1, si0, si1, ss0, ss1):
        cid = lax.axis_index("c")
        sid = lax.axis_index("s")
        wid = sid * NC + cid
        sg = (sg0, sg1)
        si = (si0, si1)
        ss = (ss0, ss1)

        pltpu.sync_copy(a2_hbm, a2_v)

        zvec = jnp.zeros((L,), jnp.float32)
        lane = lax.iota(jnp.int32, L)

        def zdn(i, c):
            dn_v[pl.ds(i * L, L)] = zvec
            return c
        lax.fori_loop(0, npad // L, zdn, 0)

        def zrow(rr, c):
            for k in range(d // L):
                rows_v[0, rr, pl.ds(k * L, L)] = zvec
            return c
        lax.fori_loop(0, blk, zrow, 0)

        rows_per_tile = npad // NS
        for c in range(rows_per_tile // blk):
            pltpu.sync_copy(rows_v.at[0, pl.ds(0, blk)],
                            agg_sh.at[pl.ds(sid * rows_per_tile
                                            + c * blk, blk)])
        plsc.subcore_barrier()

        a2v = [a2_v[pl.ds(k * L, L)] for k in range(d // L)]
        bfly = [lane ^ sh for sh in (8, 4, 2, 1)]
        zero_i = lane * 0
        shdn = [jnp.maximum(lane - sh, 0) for sh in (1, 2, 4, 8)]
        nxt = jnp.minimum(lane + 1, L - 1)

        def base_of(t):
            return wid * epw + t * blk

        def idx_descs(t, o, nrow):
            b = base_of(t)
            return (
                pltpu.make_async_copy(src_hbm.at[pl.ds(b, nrow)],
                                      idx_s.at[o, pl.ds(0, nrow)], si[o]),
                pltpu.make_async_copy(dst_hbm.at[pl.ds(b, nrow)],
                                      idx_d.at[o, pl.ds(0, nrow)], si[o]),
            )

        def gather_descs(t, o, nrow):
            b = base_of(t)
            return (
                pltpu.make_async_copy(p_hbm.at[idx_s.at[o, pl.ds(0, nrow)]],
                                      rows_v.at[o, pl.ds(0, nrow)], sg[o]),
                pltpu.make_async_copy(q_hbm.at[pl.ds(b, nrow)],
                                      q_v.at[o, pl.ds(0, nrow)], sg[o]),
                pltpu.make_async_copy(g_hbm.at[idx_d.at[o, pl.ds(0, nrow)]],
                                      gd_v.at[o, pl.ds(0, nrow)], sg[o]),
            )

        def scatter_desc(s, nrow):
            return pltpu.make_async_copy(
                rows_v.at[s, pl.ds(0, nrow)],
                agg_sh.at[idxsc.at[s, pl.ds(0, nrow)]], ss[s])

        def issue(descs):
            for cp in descs:
                cp.start()

        def drain(descs):
            for cp in descs:
                cp.wait()

        def compute_block(s, ngr):
            def grp_body(gi, c2):
                row0 = gi * L
                svec = zvec
                for j in range(L):
                    row = row0 + j
                    acc = None
                    for k in range(d // L):
                        v = rows_v[s, row, pl.ds(k * L, L)] \
                            + q_v[s, row, pl.ds(k * L, L)]
                        v = jnp.maximum(v, 0.0)
                        rows_v[s, row, pl.ds(k * L, L)] = v
                        acc = v * a2v[k] if acc is None else acc + v * a2v[k]
                    for bf in bfly:
                        acc = acc + _perm(acc, bf)
                    svec = jnp.where(lane == j, acc, svec)
                dvec = idx_d[s, pl.ds(row0, L)]
                gd = gd_v[s, pl.ds(row0, L)]
                logit = gd + svec
                logit = jnp.where(logit >= 0.0, logit, 0.2 * logit)
                w = jnp.exp(logit)
                for j in range(L):
                    row = row0 + j
                    wj = _perm(w, zero_i + j)
                    for k in range(d // L):
                        rows_v[s, row, pl.ds(k * L, L)] = \
                            rows_v[s, row, pl.ds(k * L, L)] * wj
                ds_s, ws_s = plsc.sort_key_val(dvec, w)
                for i, sh in enumerate((1, 2, 4, 8)):
                    sd = _perm(ds_s, shdn[i])
                    sw = _perm(ws_s, shdn[i])
                    take = (lane >= sh) & (sd == ds_s)
                    ws_s = ws_s + jnp.where(take, sw, 0.0)
                nd = _perm(ds_s, nxt)
                last = (ds_s != nd) | (lane == L - 1)
                plsc.addupdate_scatter(dn_v, [ds_s], ws_s, mask=last)
                return c2
            lax.fori_loop(0, ngr, grp_body, 0)
            for kk in range(blk // L):
                idxsc[s, pl.ds(kk * L, L)] = idx_d[s, pl.ds(kk * L, L)]

        b0 = base_of(0)
        pltpu.sync_copy(src_hbm.at[pl.ds(b0, blk)],
                        idx_s.at[0, pl.ds(0, blk)])
        pltpu.sync_copy(dst_hbm.at[pl.ds(b0, blk)],
                        idx_d.at[0, pl.ds(0, blk)])
        issue(gather_descs(0, 0, blk))

        def pipe_body(t2, carry):
            t0 = 2 * t2
            drain(gather_descs(t0, 0, blk))
            issue(idx_descs(t0 + 1, 1, blk))
            compute_block(0, blk // L)
            pltpu.async_copy(rows_v.at[0], agg_sh.at[idxsc.at[0]], ss0,
                             add=True)
            drain(idx_descs(t0 + 1, 1, blk))

            @pl.when(t2 > 0)
            def _():
                drain((scatter_desc(1, blk),))
            issue(gather_descs(t0 + 1, 1, blk))

            t1 = t0 + 1
            drain(gather_descs(t1, 1, blk))

            @pl.when(t2 < nhalf - 1)
            def _():
                issue(idx_descs(t1 + 1, 0, blk))
            compute_block(1, blk // L)
            pltpu.async_copy(rows_v.at[1], agg_sh.at[idxsc.at[1]], ss1,
                             add=True)

            @pl.when(t2 < nhalf - 1)
            def _():
                drain(idx_descs(t1 + 1, 0, blk))
                drain((scatter_desc(0, blk),))
                issue(gather_descs(t1 + 1, 0, blk))
            return carry
        lax.fori_loop(0, nhalf, pipe_body, 0)
        drain((scatter_desc(0, blk),))
        drain((scatter_desc(1, blk),))

        if tail:
            bt = base_of(nfull)
            pltpu.sync_copy(src_hbm.at[pl.ds(bt, tail)],
                            idx_s.at[0, pl.ds(0, tail)])
            pltpu.sync_copy(dst_hbm.at[pl.ds(bt, tail)],
                            idx_d.at[0, pl.ds(0, tail)])
            tail_g = gather_descs(nfull, 0, tail)
            issue(tail_g)
            drain(tail_g)
            compute_block(0, tail // L)
            pltpu.async_copy(rows_v.at[0, pl.ds(0, tail)],
                             agg_sh.at[idxsc.at[0, pl.ds(0, tail)]], ss0,
                             add=True)
            drain((scatter_desc(0, tail),))

        plsc.subcore_barrier()
        for c in range(rows_per_tile // blk):
            r0 = sid * rows_per_tile + c * blk
            pltpu.sync_copy(agg_sh.at[pl.ds(r0, blk)],
                            rows_v.at[0, pl.ds(0, blk)])
            pltpu.sync_copy(rows_v.at[0, pl.ds(0, blk)],
                            agg_hbm.at[cid, pl.ds(r0, blk)])
        pltpu.sync_copy(dn_v, dn_hbm.at[cid, sid])

    return edge_kernel


def kernel(x, edge_attr, W_edge, a_att, W_z, b_z, W_r, b_r, W_h, b_h,
           edge_index):
    n, d = x.shape
    e, de = edge_attr.shape
    blk = 64
    align = NS * blk
    npad = ((n + align - 1) // align) * align
    rblk = 400
    qblk = 4000

    src = edge_index[0].astype(jnp.int32)
    dst = edge_index[1].astype(jnp.int32)
    w1 = W_edge[:d]
    w2 = W_edge[d:]
    a1 = a_att[:d]
    a2 = a_att[d:, 0]

    p_mat, g_mat = pl.pallas_call(
        _pre_body,
        grid=(n // rblk,),
        in_specs=[
            pl.BlockSpec((rblk, d), lambda i: (i, 0)),
            pl.BlockSpec((d, d), lambda i: (0, 0)),
            pl.BlockSpec((d, 1), lambda i: (0, 0)),
        ],
        out_specs=[
            pl.BlockSpec((rblk, d), lambda i: (i, 0)),
            pl.BlockSpec((rblk, 1), lambda i: (i, 0)),
        ],
        out_shape=[
            jax.ShapeDtypeStruct((n, d), jnp.float32),
            jax.ShapeDtypeStruct((n, 1), jnp.float32),
        ],
    )(x, w1, a1)
    g_vec = g_mat.reshape(n)

    q_mat = pl.pallas_call(
        _q_body,
        grid=(e // qblk,),
        in_specs=[
            pl.BlockSpec((qblk, de), lambda i: (i, 0)),
            pl.BlockSpec((de, d), lambda i: (0, 0)),
        ],
        out_specs=pl.BlockSpec((qblk, d), lambda i: (i, 0)),
        out_shape=jax.ShapeDtypeStruct((e, d), jnp.float32),
    )(edge_attr, w2)

    edge_kernel = _make_edge_kernel(n, e, d, npad, blk)
    agg_parts, dn_parts = edge_kernel(p_mat, q_mat, g_vec, src, dst, a2)

    dsum = pl.pallas_call(
        _dsum_body,
        in_specs=[pl.BlockSpec((NW, npad), lambda: (0, 0))],
        out_specs=pl.BlockSpec((1, npad), lambda: (0, 0)),
        out_shape=jax.ShapeDtypeStruct((1, npad), jnp.float32),
    )(dn_parts.reshape(NW, npad))
    dcol = dsum.reshape(npad)[:n].reshape(n, 1)

    out = pl.pallas_call(
        _post_body,
        grid=(n // rblk,),
        in_specs=[
            pl.BlockSpec((NC, rblk, d), lambda i: (0, i, 0)),
            pl.BlockSpec((rblk, 1), lambda i: (i, 0)),
            pl.BlockSpec((rblk, d), lambda i: (i, 0)),
            pl.BlockSpec((d, d), lambda i: (0, 0)),
            pl.BlockSpec((d, d), lambda i: (0, 0)),
            pl.BlockSpec((d, d), lambda i: (0, 0)),
            pl.BlockSpec((d, d), lambda i: (0, 0)),
            pl.BlockSpec((d, d), lambda i: (0, 0)),
            pl.BlockSpec((d, d), lambda i: (0, 0)),
            pl.BlockSpec((1, d), lambda i: (0, 0)),
            pl.BlockSpec((1, d), lambda i: (0, 0)),
            pl.BlockSpec((1, d), lambda i: (0, 0)),
        ],
        out_specs=pl.BlockSpec((rblk, d), lambda i: (i, 0)),
        out_shape=jax.ShapeDtypeStruct((n, d), jnp.float32),
    )(agg_parts, dcol, x, W_z[:d], W_z[d:], W_r[:d], W_r[d:], W_h[:d],
      W_h[d:], b_z.reshape(1, d), b_r.reshape(1, d), b_h.reshape(1, d))
    return out
